# Initial kernel scaffold; baseline (speedup 1.0000x reference)
import jax
import jax.numpy as jnp
from jax.experimental import pallas as pl


def kernel(previous_inclusion_score, nodes, adjacency_matrix, W_phi, W_theta):
    raise NotImplementedError("write your pallas kernel here")


# speedup vs baseline: 1.4851x; 1.4851x over previous
import functools

import jax
import jax.numpy as jnp
from jax.experimental import pallas as pl

N = 8192
BI = 256
BJ = 2048


def _body(row_ref, col_ref, adj_ref, out_ref, *, nj):
    j = pl.program_id(1)

    sqi = row_ref[:, 0:1]
    x0i = row_ref[:, 1:2]
    x1i = row_ref[:, 2:3]
    x2i = row_ref[:, 3:4]

    sqj = col_ref[0:1, :]
    x0j = col_ref[1:2, :]
    x1j = col_ref[2:3, :]
    x2j = col_ref[3:4, :]

    g = x0i * x0j + x1i * x1j + x2i * x2j
    d2 = (sqi + sqj) - 2.0 * g
    d2m = jnp.where(adj_ref[:, :] != 0, d2, -jnp.inf)
    m = jnp.max(d2m, axis=1)

    @pl.when(j == 0)
    def _():
        out_ref[0, :] = m

    @pl.when(j > 0)
    def _():
        out_ref[0, :] = jnp.maximum(out_ref[0, :], m)

    @pl.when(j == nj - 1)
    def _():
        acc = out_ref[0, :]
        maxd = jnp.sqrt(jnp.maximum(acc, 0.0))
        prev = row_ref[:, 4]
        phimean = row_ref[:, 5]
        out_ref[0, :] = 0.5 * (prev + maxd * phimean)


@jax.jit
def kernel(previous_inclusion_score, nodes, adjacency_matrix, W_phi, W_theta):
    w = W_theta[:, 0]
    wx = nodes * w[None, :]
    sq = jnp.sum(wx * wx, axis=1)
    phimean = jnp.mean(W_phi)

    zeros = jnp.zeros((N,), jnp.float32)
    cols = jnp.stack(
        [sq, wx[:, 0], wx[:, 1], wx[:, 2],
         previous_inclusion_score, jnp.full((N,), phimean),
         zeros, zeros], axis=0)
    rows = cols.T

    ni = N // BI
    nj = N // BJ
    out = pl.pallas_call(
        functools.partial(_body, nj=nj),
        grid=(ni, nj),
        in_specs=[
            pl.BlockSpec((BI, 8), lambda i, j: (i, 0)),
            pl.BlockSpec((8, BJ), lambda i, j: (0, j)),
            pl.BlockSpec((BI, BJ), lambda i, j: (i, j)),
        ],
        out_specs=pl.BlockSpec((1, BI), lambda i, j: (0, i)),
        out_shape=jax.ShapeDtypeStruct((1, N), jnp.float32),
    )(rows, cols, adjacency_matrix)
    return out[0]


# speedup vs baseline: 1.6425x; 1.1060x over previous
import functools

import jax
import jax.numpy as jnp
from jax.experimental import pallas as pl

N = 8192
BI = 256
BJ = 2048


def _body(row_ref, col_ref, adj_ref, out_ref, *, nj):
    j = pl.program_id(1)

    t = jnp.dot(row_ref[:, :], col_ref[:, :],
                preferred_element_type=jnp.float32)
    tm = jnp.where(adj_ref[:, :] != 0, t, -jnp.inf)
    m = jnp.max(tm, axis=1)

    @pl.when(j == 0)
    def _():
        out_ref[0, :] = m

    @pl.when(j > 0)
    def _():
        out_ref[0, :] = jnp.maximum(out_ref[0, :], m)

    @pl.when(j == nj - 1)
    def _():
        acc = out_ref[0, :] + row_ref[:, 6]
        maxd = jnp.sqrt(jnp.maximum(acc, 0.0))
        prev = row_ref[:, 4]
        phimean = row_ref[:, 5]
        out_ref[0, :] = 0.5 * (prev + maxd * phimean)


@jax.jit
def kernel(previous_inclusion_score, nodes, adjacency_matrix, W_phi, W_theta):
    w = W_theta[:, 0]
    wx = nodes * w[None, :]
    sq = jnp.sum(wx * wx, axis=1)
    phimean = jnp.mean(W_phi)

    zeros = jnp.zeros((N,), jnp.float32)
    ones = jnp.ones((N,), jnp.float32)
    rows = jnp.stack(
        [wx[:, 0], wx[:, 1], wx[:, 2], ones,
         previous_inclusion_score, jnp.full((N,), phimean),
         sq, zeros], axis=1)
    cols = jnp.stack(
        [-2.0 * wx[:, 0], -2.0 * wx[:, 1], -2.0 * wx[:, 2], sq,
         zeros, zeros, zeros, zeros], axis=0)

    ni = N // BI
    nj = N // BJ
    out = pl.pallas_call(
        functools.partial(_body, nj=nj),
        grid=(ni, nj),
        in_specs=[
            pl.BlockSpec((BI, 8), lambda i, j: (i, 0)),
            pl.BlockSpec((8, BJ), lambda i, j: (0, j)),
            pl.BlockSpec((BI, BJ), lambda i, j: (i, j)),
        ],
        out_specs=pl.BlockSpec((1, BI), lambda i, j: (0, i)),
        out_shape=jax.ShapeDtypeStruct((1, N), jnp.float32),
    )(rows, cols, adjacency_matrix)
    return out[0]


# speedup vs baseline: 1.8078x; 1.1006x over previous
import functools

import jax
import jax.numpy as jnp
from jax.experimental import pallas as pl
from jax.experimental.pallas import tpu as pltpu

N = 8192
BI = 256
BJ = 2048
NEG = float("-inf")


def _body(row_ref, col_ref, adj_ref, out_ref, acc_ref, *, nj):
    j = pl.program_id(1)

    t = jnp.dot(row_ref[:, :], col_ref[:, :],
                preferred_element_type=jnp.float32)
    adj = adj_ref[:, :]

    part = None
    for c in range(BJ // 128):
        sl = slice(c * 128, (c + 1) * 128)
        piece = jnp.where(adj[:, sl] != 0, t[:, sl], NEG)
        part = piece if part is None else jnp.maximum(part, piece)

    @pl.when(j == 0)
    def _():
        acc_ref[:, :] = part

    @pl.when(j > 0)
    def _():
        acc_ref[:, :] = jnp.maximum(acc_ref[:, :], part)

    @pl.when(j == nj - 1)
    def _():
        acc = jnp.max(acc_ref[:, :], axis=1, keepdims=True)
        d2 = acc + row_ref[:, 6:7]
        maxd = jnp.sqrt(jnp.maximum(d2, 0.0))
        prev = row_ref[:, 4:5]
        phimean = row_ref[:, 5:6]
        out_ref[:, :] = 0.5 * (prev + maxd * phimean)


@jax.jit
def kernel(previous_inclusion_score, nodes, adjacency_matrix, W_phi, W_theta):
    w = W_theta[:, 0]
    wx = nodes * w[None, :]
    sq = jnp.sum(wx * wx, axis=1)
    phimean = jnp.mean(W_phi)

    zeros = jnp.zeros((N,), jnp.float32)
    ones = jnp.ones((N,), jnp.float32)
    rows = jnp.stack(
        [wx[:, 0], wx[:, 1], wx[:, 2], ones,
         previous_inclusion_score, jnp.full((N,), phimean),
         sq, zeros], axis=1)
    cols = jnp.stack(
        [-2.0 * wx[:, 0], -2.0 * wx[:, 1], -2.0 * wx[:, 2], sq,
         zeros, zeros, zeros, zeros], axis=0)

    ni = N // BI
    nj = N // BJ
    out = pl.pallas_call(
        functools.partial(_body, nj=nj),
        grid=(ni, nj),
        in_specs=[
            pl.BlockSpec((BI, 8), lambda i, j: (i, 0)),
            pl.BlockSpec((8, BJ), lambda i, j: (0, j)),
            pl.BlockSpec((BI, BJ), lambda i, j: (i, j)),
        ],
        out_specs=pl.BlockSpec((BI, 1), lambda i, j: (i, 0)),
        out_shape=jax.ShapeDtypeStruct((N, 1), jnp.float32),
        scratch_shapes=[pltpu.VMEM((BI, 128), jnp.float32)],
    )(rows, cols, adjacency_matrix)
    return out[:, 0]
